# Initial kernel scaffold; baseline (speedup 1.0000x reference)
import jax
import jax.numpy as jnp
from jax.experimental import pallas as pl


def kernel(x, edge_index, W1, b1, W2, b2):
    raise NotImplementedError("write your pallas kernel here")


# speedup vs baseline: 3.0269x; 3.0269x over previous
import jax
import jax.numpy as jnp
from jax.experimental import pallas as pl

N = 50000
F = 1433
H = 64
C = 7
E = 800000
FP = 1536


def _mm_body(x_ref, w_ref, o_ref):
    o_ref[...] = jnp.dot(x_ref[...], w_ref[...], preferred_element_type=jnp.float32)


def _matmul(x, w):
    m_blk = 2000
    return pl.pallas_call(
        _mm_body,
        grid=(N // m_blk,),
        in_specs=[
            pl.BlockSpec((m_blk, FP), lambda i: (i, 0)),
            pl.BlockSpec((FP, H), lambda i: (0, 0)),
        ],
        out_specs=pl.BlockSpec((m_blk, H), lambda i: (i, 0)),
        out_shape=jax.ShapeDtypeStruct((N, H), jnp.float32),
    )(x, w)


def kernel(x, edge_index, W1, b1, W2, b2):
    src = edge_index[0]
    dst = edge_index[1]
    deg = jnp.zeros((N,), jnp.float32).at[dst].add(1.0) + 1.0
    dinv = jax.lax.rsqrt(deg)

    xp = jnp.pad(x, ((0, 0), (0, FP - F)))
    w1p = jnp.pad(W1, ((0, FP - F), (0, 0)))
    xw = _matmul(xp, w1p)

    y1 = xw * dinv[:, None]
    acc = jnp.zeros((N, H), jnp.float32).at[dst].add(y1[src])
    h = jax.nn.relu(dinv[:, None] * (acc + y1) + b1)

    hw = h @ W2
    y2 = hw * dinv[:, None]
    acc2 = jnp.zeros((N, C), jnp.float32).at[dst].add(y2[src])
    out = dinv[:, None] * (acc2 + y2) + b2
    return jax.nn.log_softmax(out, axis=1)


# speedup vs baseline: 9.4241x; 3.1134x over previous
import functools

import jax
import jax.numpy as jnp
from jax import lax
from jax.experimental import pallas as pl
from jax.experimental.pallas import tpu as pltpu
from jax.experimental.pallas import tpu_sc as plsc

N = 50000
F = 1433
H = 64
C = 7
E = 800000

FP = 1536
NPAD = 50048
EPAD = 819200
NCHUNK = EPAD // 128
NC, NS = 2, 16
MBLK = 3128

_mesh = plsc.VectorSubcoreMesh(
    core_axis_name="c", subcore_axis_name="s", num_cores=NC, num_subcores=NS
)
_sc_params = pltpu.CompilerParams(use_tc_tiling_on_sc=False)


@functools.partial(
    pl.kernel,
    out_type=jax.ShapeDtypeStruct((NC, NPAD, 8), jnp.float32),
    mesh=_mesh,
    compiler_params=_sc_params,
    scratch_types=[
        pltpu.VMEM((8, 128), jnp.int32),
        pltpu.VMEM((128, 8), jnp.float32),
        pltpu.VMEM_SHARED((NPAD, 8), jnp.float32),
        pltpu.SemaphoreType.DMA,
    ],
)
def _k1_deg(dst2_hbm, ones_hbm, zeros_hbm, degp_hbm, idx_v, ones_v, acc_sh, sem):
    c = lax.axis_index("c")
    s = lax.axis_index("s")
    rbase = s * (NPAD // NS)
    pltpu.sync_copy(zeros_hbm.at[pl.ds(rbase, NPAD // NS)],
                    acc_sh.at[pl.ds(rbase, NPAD // NS)])
    pltpu.sync_copy(ones_hbm, ones_v)
    plsc.subcore_barrier()

    row0 = c * (NCHUNK // NC) + s * (NCHUNK // NC // NS)

    def blk(b, carry):
        crow = row0 + b * 8
        pltpu.sync_copy(dst2_hbm.at[pl.ds(crow, 8)], idx_v)
        for j in range(8):
            pltpu.sync_copy(ones_v, acc_sh.at[idx_v.at[j]], add=True)
        return carry

    lax.fori_loop(0, NCHUNK // NC // NS // 8, blk, 0)
    plsc.subcore_barrier()
    pltpu.sync_copy(acc_sh.at[pl.ds(rbase, NPAD // NS)],
                    degp_hbm.at[c].at[pl.ds(rbase, NPAD // NS)])


@functools.partial(
    pl.kernel,
    out_type=jax.ShapeDtypeStruct((NC, NPAD, 32), jnp.float32),
    mesh=_mesh,
    compiler_params=_sc_params,
    scratch_types=[
        pltpu.VMEM((16, 128), jnp.int32),
        pltpu.VMEM((16, 128), jnp.int32),
        pltpu.VMEM((128, 32), jnp.float32),
        pltpu.VMEM((128, 32), jnp.float32),
        pltpu.VMEM_SHARED((NPAD, 32), jnp.float32),
        pltpu.SemaphoreType.DMA,
        pltpu.SemaphoreType.DMA,
    ],
)
def _k2_agg64(y_hbm, src2_hbm, dst2_hbm, out_hbm,
              sidx_v, didx_v, rows_a, rows_b, acc_sh, sem_a, sem_b):
    c = lax.axis_index("c")
    s = lax.axis_index("s")
    rbase = s * (NPAD // NS)
    pltpu.sync_copy(y_hbm.at[c].at[pl.ds(rbase, NPAD // NS)],
                    acc_sh.at[pl.ds(rbase, NPAD // NS)])
    plsc.subcore_barrier()

    row0 = s * (NCHUNK // NS)
    bufs = (rows_a, rows_b)
    sems = (sem_a, sem_b)

    def blk(b, carry):
        crow = row0 + b * 16
        pltpu.sync_copy(src2_hbm.at[pl.ds(crow, 16)], sidx_v)
        pltpu.sync_copy(dst2_hbm.at[pl.ds(crow, 16)], didx_v)
        cps = [None, None]
        cps[0] = pltpu.async_copy(y_hbm.at[c].at[sidx_v.at[0]], bufs[0], sems[0])
        for j in range(16):
            p = j % 2
            cps[p].wait()
            if j < 15:
                q = (j + 1) % 2
                cps[q] = pltpu.async_copy(
                    y_hbm.at[c].at[sidx_v.at[j + 1]], bufs[q], sems[q])
            pltpu.sync_copy(bufs[p], acc_sh.at[didx_v.at[j]], add=True)
        return carry

    lax.fori_loop(0, NCHUNK // NS // 16, blk, 0)
    plsc.subcore_barrier()
    pltpu.sync_copy(acc_sh.at[pl.ds(rbase, NPAD // NS)],
                    out_hbm.at[c].at[pl.ds(rbase, NPAD // NS)])


@functools.partial(
    pl.kernel,
    out_type=jax.ShapeDtypeStruct((NC, NPAD, 16), jnp.float32),
    mesh=_mesh,
    compiler_params=_sc_params,
    scratch_types=[
        pltpu.VMEM((8, 128), jnp.int32),
        pltpu.VMEM((8, 128), jnp.int32),
        pltpu.VMEM((128, 16), jnp.float32),
        pltpu.VMEM((128, 16), jnp.float32),
        pltpu.VMEM_SHARED((NPAD, 16), jnp.float32),
        pltpu.SemaphoreType.DMA,
        pltpu.SemaphoreType.DMA,
    ],
)
def _k3_agg16(y2_hbm, zeros_hbm, src2_hbm, dst2_hbm, out_hbm,
              sidx_v, didx_v, rows_a, rows_b, acc_sh, sem_a, sem_b):
    c = lax.axis_index("c")
    s = lax.axis_index("s")
    rbase = s * (NPAD // NS)
    nsl = NPAD // NS

    @pl.when(c == 0)
    def _():
        pltpu.sync_copy(y2_hbm.at[pl.ds(rbase, nsl)], acc_sh.at[pl.ds(rbase, nsl)])

    @pl.when(c != 0)
    def _():
        pltpu.sync_copy(zeros_hbm.at[pl.ds(rbase, nsl)], acc_sh.at[pl.ds(rbase, nsl)])

    plsc.subcore_barrier()

    row0 = c * (NCHUNK // NC) + s * (NCHUNK // NC // NS)
    bufs = (rows_a, rows_b)
    sems = (sem_a, sem_b)

    def blk(b, carry):
        crow = row0 + b * 8
        pltpu.sync_copy(src2_hbm.at[pl.ds(crow, 8)], sidx_v)
        pltpu.sync_copy(dst2_hbm.at[pl.ds(crow, 8)], didx_v)
        cps = [None, None]
        cps[0] = pltpu.async_copy(y2_hbm.at[sidx_v.at[0]], bufs[0], sems[0])
        for j in range(8):
            p = j % 2
            cps[p].wait()
            if j < 7:
                q = (j + 1) % 2
                cps[q] = pltpu.async_copy(
                    y2_hbm.at[sidx_v.at[j + 1]], bufs[q], sems[q])
            pltpu.sync_copy(bufs[p], acc_sh.at[didx_v.at[j]], add=True)
        return carry

    lax.fori_loop(0, NCHUNK // NC // NS // 8, blk, 0)
    plsc.subcore_barrier()
    pltpu.sync_copy(acc_sh.at[pl.ds(rbase, nsl)],
                    out_hbm.at[c].at[pl.ds(rbase, nsl)])


def _mm_body(x_ref, w_ref, o_ref):
    o_ref[...] = jnp.dot(x_ref[...], w_ref[...], preferred_element_type=jnp.float32)


def _matmul(x, w):
    m_blk = 2000
    return pl.pallas_call(
        _mm_body,
        grid=(N // m_blk,),
        in_specs=[
            pl.BlockSpec((m_blk, FP), lambda i: (i, 0)),
            pl.BlockSpec((FP, H), lambda i: (0, 0)),
        ],
        out_specs=pl.BlockSpec((m_blk, H), lambda i: (i, 0)),
        out_shape=jax.ShapeDtypeStruct((N, H), jnp.float32),
    )(x, w)


def _s1_body(xw_ref, degp_ref, y_ref, dinv_ref):
    deg8 = degp_ref[0] + degp_ref[1] + 1.0
    dinv8 = lax.rsqrt(deg8)
    dinv_ref[...] = dinv8
    y = xw_ref[...] * dinv8[:, 0:1]
    y_ref[0] = y[:, :32]
    y_ref[1] = y[:, 32:]


def _s1_scale(xw, degp):
    return pl.pallas_call(
        _s1_body,
        grid=(NPAD // MBLK,),
        in_specs=[
            pl.BlockSpec((MBLK, H), lambda i: (i, 0)),
            pl.BlockSpec((NC, MBLK, 8), lambda i: (0, i, 0)),
        ],
        out_specs=[
            pl.BlockSpec((NC, MBLK, 32), lambda i: (0, i, 0)),
            pl.BlockSpec((MBLK, 8), lambda i: (i, 0)),
        ],
        out_shape=[
            jax.ShapeDtypeStruct((NC, NPAD, 32), jnp.float32),
            jax.ShapeDtypeStruct((NPAD, 8), jnp.float32),
        ],
    )(xw, degp)


def _s2_body(acc_ref, dinv_ref, w2_ref, b1_ref, y2_ref):
    acc = jnp.concatenate([acc_ref[0], acc_ref[1]], axis=1)
    dinv = dinv_ref[:, 0:1]
    h = jax.nn.relu(acc * dinv + b1_ref[...])
    hw = jnp.dot(h, w2_ref[...], preferred_element_type=jnp.float32)
    y2_ref[...] = hw * dinv


def _s2_hidden(acc, dinv8, w2p, b1):
    return pl.pallas_call(
        _s2_body,
        grid=(NPAD // MBLK,),
        in_specs=[
            pl.BlockSpec((NC, MBLK, 32), lambda i: (0, i, 0)),
            pl.BlockSpec((MBLK, 8), lambda i: (i, 0)),
            pl.BlockSpec((H, 16), lambda i: (0, 0)),
            pl.BlockSpec((1, H), lambda i: (0, 0)),
        ],
        out_specs=pl.BlockSpec((MBLK, 16), lambda i: (i, 0)),
        out_shape=jax.ShapeDtypeStruct((NPAD, 16), jnp.float32),
    )(acc, dinv8, w2p, b1)


def _s3_body(accp_ref, dinv_ref, b2_ref, o_ref):
    acc = accp_ref[0] + accp_ref[1]
    logits = acc * dinv_ref[:, 0:1] + b2_ref[...]
    x7 = logits[:, :C]
    m = jnp.max(x7, axis=1, keepdims=True)
    lse = jnp.log(jnp.sum(jnp.exp(x7 - m), axis=1, keepdims=True)) + m
    o_ref[...] = jnp.pad(x7 - lse, ((0, 0), (0, 16 - C)))


def _s3_out(accp, dinv8, b2p):
    return pl.pallas_call(
        _s3_body,
        grid=(NPAD // MBLK,),
        in_specs=[
            pl.BlockSpec((NC, MBLK, 16), lambda i: (0, i, 0)),
            pl.BlockSpec((MBLK, 8), lambda i: (i, 0)),
            pl.BlockSpec((1, 16), lambda i: (0, 0)),
        ],
        out_specs=pl.BlockSpec((MBLK, 16), lambda i: (i, 0)),
        out_shape=jax.ShapeDtypeStruct((NPAD, 16), jnp.float32),
    )(accp, dinv8, b2p)


def kernel(x, edge_index, W1, b1, W2, b2):
    src = edge_index[0].astype(jnp.int32)
    dst = edge_index[1].astype(jnp.int32)
    src2 = jnp.concatenate(
        [src, jnp.zeros((EPAD - E,), jnp.int32)]).reshape(NCHUNK, 128)
    dst2 = jnp.concatenate(
        [dst, jnp.full((EPAD - E,), NPAD - 1, jnp.int32)]).reshape(NCHUNK, 128)

    ones8 = jnp.ones((128, 8), jnp.float32)
    zeros8 = jnp.zeros((NPAD, 8), jnp.float32)
    zeros16 = jnp.zeros((NPAD, 16), jnp.float32)

    degp = _k1_deg(dst2, ones8, zeros8)

    xp = jnp.pad(x, ((0, 0), (0, FP - F)))
    w1p = jnp.pad(W1, ((0, FP - F), (0, 0)))
    xw = jnp.pad(_matmul(xp, w1p), ((0, NPAD - N), (0, 0)))

    y_stack, dinv8 = _s1_scale(xw, degp)
    acc_stack = _k2_agg64(y_stack, src2, dst2)

    w2p = jnp.pad(W2, ((0, 0), (0, 16 - C)))
    y2 = _s2_hidden(acc_stack, dinv8, w2p, b1.reshape(1, H))
    accp2 = _k3_agg16(y2, zeros16, src2, dst2)

    b2p = jnp.pad(b2, (0, 16 - C)).reshape(1, 16)
    outp = _s3_out(accp2, dinv8, b2p)
    return outp[:N, :C]


# speedup vs baseline: 10.4484x; 1.1087x over previous
import functools

import jax
import jax.numpy as jnp
from jax import lax
from jax.experimental import pallas as pl
from jax.experimental.pallas import tpu as pltpu
from jax.experimental.pallas import tpu_sc as plsc

N = 50000
F = 1433
H = 64
C = 7
E = 800000

FP = 1536
NROW = 392
NPAD = NROW * 128
EPAD = 819200
NCHUNK = EPAD // 128
NC, NS = 2, 16
NSL = NPAD // NS
MBLK = NPAD // 8

_mesh = plsc.VectorSubcoreMesh(
    core_axis_name="c", subcore_axis_name="s", num_cores=NC, num_subcores=NS
)
_sc_params = pltpu.CompilerParams(use_tc_tiling_on_sc=False)
_sc_params_nolayout = pltpu.CompilerParams(
    use_tc_tiling_on_sc=False, needs_layout_passes=False)


@functools.partial(
    pl.kernel,
    out_type=jax.ShapeDtypeStruct((NC * NS, NROW, 128), jnp.float32),
    mesh=_mesh,
    compiler_params=_sc_params_nolayout,
    scratch_types=[
        pltpu.VMEM((8, 128), jnp.int32),
        pltpu.VMEM((NROW, 128), jnp.float32),
    ],
)
def _k1_deg(dst2_hbm, hist_hbm, idx_v, hist_v):
    c = lax.axis_index("c")
    s = lax.axis_index("s")
    wid = c * NS + s

    zeros = jnp.zeros((16,), jnp.float32)

    def zrow(r, carry):
        for k in range(8):
            hist_v[r, pl.ds(k * 16, 16)] = zeros
        return carry

    lax.fori_loop(0, NROW, zrow, 0)

    ones = jnp.ones((16,), jnp.float32)
    row0 = wid * (NCHUNK // (NC * NS))

    def blk(b, carry):
        crow = row0 + b * 8
        pltpu.sync_copy(dst2_hbm.at[pl.ds(crow, 8)], idx_v)
        for j in range(8):
            for k in range(8):
                dvec = idx_v[j, pl.ds(k * 16, 16)]
                r = lax.shift_right_logical(dvec, 7)
                cc = lax.bitwise_and(dvec, 127)
                plsc.addupdate_scatter(hist_v, [r, cc], ones)
        return carry

    lax.fori_loop(0, NCHUNK // (NC * NS) // 8, blk, 0)
    pltpu.sync_copy(hist_v, hist_hbm.at[wid])


@functools.partial(
    pl.kernel,
    out_type=[
        jax.ShapeDtypeStruct((NPAD, 128), jnp.float32),
        jax.ShapeDtypeStruct((NC, NPAD, 16), jnp.float32),
    ],
    mesh=_mesh,
    compiler_params=_sc_params,
    scratch_types=[
        pltpu.VMEM((4, 128), jnp.int32),
        pltpu.VMEM((4, 128), jnp.int32),
        pltpu.VMEM((4, 128), jnp.int32),
        pltpu.VMEM((4, 128), jnp.int32),
        pltpu.VMEM((4, 128, 16), jnp.float32),
        pltpu.VMEM((4, 128, 16), jnp.float32),
        pltpu.VMEM((NROW, 16), jnp.float32),
        pltpu.VMEM_SHARED((NPAD, 16), jnp.float32),
        pltpu.SemaphoreType.DMA((4,)),
        pltpu.SemaphoreType.DMA((4,)),
        pltpu.SemaphoreType.DMA((4,)),
        pltpu.SemaphoreType.DMA((4,)),
    ],
)
def _k2_agg64(y_hbm, src2_hbm, dst2_hbm, out_hbm, ycomp_hbm,
              sidx_a, didx_a, sidx_b, didx_b, bufs_a, bufs_b, stage_v, acc_sh,
              gsem_a, ssem_a, gsem_b, ssem_b):
    c = lax.axis_index("c")
    s = lax.axis_index("s")
    rbase = s * NSL
    rows = pl.ds(rbase, NSL)
    row0 = s * (NCHUNK // NS)
    tab = ycomp_hbm.at[c]

    for p in range(2):
        def compact(i, carry, p=p):
            rr = pl.ds(rbase + i * NROW, NROW)

            @pl.when(c == 0)
            def _():
                pltpu.sync_copy(y_hbm.at[rr, pl.ds(16 * p, 16)], stage_v)

            @pl.when(c != 0)
            def _():
                pltpu.sync_copy(y_hbm.at[rr, pl.ds(32 + 16 * p, 16)], stage_v)

            pltpu.sync_copy(stage_v, ycomp_hbm.at[c].at[rr])
            pltpu.sync_copy(stage_v, acc_sh.at[rr])
            return carry

        lax.fori_loop(0, NSL // NROW, compact, 0)
        plsc.subcore_barrier()

        def halfstep_issue(crow, sidx, didx, bufs, gsem, ssem, drain):
            if drain:
                for j in range(4):
                    pltpu.make_async_copy(
                        bufs.at[j], acc_sh.at[didx.at[j]], ssem.at[j]).wait()
            pltpu.sync_copy(src2_hbm.at[pl.ds(crow, 4)], sidx)
            pltpu.sync_copy(dst2_hbm.at[pl.ds(crow, 4)], didx)
            gds = []
            for j in range(4):
                d = pltpu.make_async_copy(tab.at[sidx.at[j]], bufs.at[j],
                                          gsem.at[j])
                d.start()
                gds.append(d)
            return gds

        def halfstep_finish(gds, didx, bufs, ssem):
            for j in range(4):
                gds[j].wait()
                d = pltpu.make_async_copy(bufs.at[j], acc_sh.at[didx.at[j]],
                                          ssem.at[j])
                d.start(add=True)

        def blk(b, carry):
            crow = row0 + b * 8

            @pl.when(b == 0)
            def _():
                ga = halfstep_issue(crow, sidx_a, didx_a, bufs_a, gsem_a,
                                    ssem_a, False)
                gb = halfstep_issue(crow + 4, sidx_b, didx_b, bufs_b, gsem_b,
                                    ssem_b, False)
                halfstep_finish(ga, didx_a, bufs_a, ssem_a)
                halfstep_finish(gb, didx_b, bufs_b, ssem_b)

            @pl.when(b != 0)
            def _():
                ga = halfstep_issue(crow, sidx_a, didx_a, bufs_a, gsem_a,
                                    ssem_a, True)
                gb = halfstep_issue(crow + 4, sidx_b, didx_b, bufs_b, gsem_b,
                                    ssem_b, True)
                halfstep_finish(ga, didx_a, bufs_a, ssem_a)
                halfstep_finish(gb, didx_b, bufs_b, ssem_b)

            return carry

        lax.fori_loop(0, NCHUNK // NS // 8, blk, 0)
        for j in range(4):
            pltpu.make_async_copy(bufs_a.at[j], acc_sh.at[didx_a.at[j]],
                                  ssem_a.at[j]).wait()
            pltpu.make_async_copy(bufs_b.at[j], acc_sh.at[didx_b.at[j]],
                                  ssem_b.at[j]).wait()
        plsc.subcore_barrier()

        @pl.when(c == 0)
        def _(p=p):
            pltpu.sync_copy(acc_sh.at[rows],
                            out_hbm.at[rows, pl.ds(16 * p, 16)])

        @pl.when(c != 0)
        def _(p=p):
            pltpu.sync_copy(acc_sh.at[rows],
                            out_hbm.at[rows, pl.ds(32 + 16 * p, 16)])

        plsc.subcore_barrier()


@functools.partial(
    pl.kernel,
    out_type=[
        jax.ShapeDtypeStruct((NPAD, 128), jnp.float32),
        jax.ShapeDtypeStruct((NC, NPAD, 8), jnp.float32),
    ],
    mesh=_mesh,
    compiler_params=_sc_params,
    scratch_types=[
        pltpu.VMEM((4, 128), jnp.int32),
        pltpu.VMEM((4, 128), jnp.int32),
        pltpu.VMEM((4, 128), jnp.int32),
        pltpu.VMEM((4, 128), jnp.int32),
        pltpu.VMEM((4, 128, 8), jnp.float32),
        pltpu.VMEM((4, 128, 8), jnp.float32),
        pltpu.VMEM((NROW, 8), jnp.float32),
        pltpu.VMEM_SHARED((NPAD, 8), jnp.float32),
        pltpu.SemaphoreType.DMA((4,)),
        pltpu.SemaphoreType.DMA((4,)),
        pltpu.SemaphoreType.DMA((4,)),
        pltpu.SemaphoreType.DMA((4,)),
    ],
)
def _k3_agg16(y2_hbm, zeros_hbm, src2_hbm, dst2_hbm, out_hbm, y2c_hbm,
              sidx_a, didx_a, sidx_b, didx_b, bufs_a, bufs_b, stage_v, acc_sh,
              gsem_a, ssem_a, gsem_b, ssem_b):
    c = lax.axis_index("c")
    s = lax.axis_index("s")
    rbase = s * NSL
    rows = pl.ds(rbase, NSL)

    def compact(i, carry):
        rr = pl.ds(rbase + i * NROW, NROW)
        pltpu.sync_copy(y2_hbm.at[rr, pl.ds(0, 8)], stage_v)
        pltpu.sync_copy(stage_v, y2c_hbm.at[c].at[rr])

        @pl.when(c == 0)
        def _():
            pltpu.sync_copy(stage_v, acc_sh.at[rr])

        return carry

    lax.fori_loop(0, NSL // NROW, compact, 0)

    @pl.when(c != 0)
    def _():
        pltpu.sync_copy(zeros_hbm.at[rows], acc_sh.at[rows])

    plsc.subcore_barrier()

    tab = y2c_hbm.at[c]
    row0 = c * (NCHUNK // NC) + s * (NCHUNK // NC // NS)

    def halfstep_issue(crow, sidx, didx, bufs, gsem, ssem, drain):
        if drain:
            for j in range(4):
                pltpu.make_async_copy(
                    bufs.at[j], acc_sh.at[didx.at[j]], ssem.at[j]).wait()
        pltpu.sync_copy(src2_hbm.at[pl.ds(crow, 4)], sidx)
        pltpu.sync_copy(dst2_hbm.at[pl.ds(crow, 4)], didx)
        gds = []
        for j in range(4):
            d = pltpu.make_async_copy(tab.at[sidx.at[j]], bufs.at[j], gsem.at[j])
            d.start()
            gds.append(d)
        return gds

    def halfstep_finish(gds, didx, bufs, ssem):
        for j in range(4):
            gds[j].wait()
            d = pltpu.make_async_copy(bufs.at[j], acc_sh.at[didx.at[j]],
                                      ssem.at[j])
            d.start(add=True)

    def blk(b, carry):
        crow = row0 + b * 8

        @pl.when(b == 0)
        def _():
            ga = halfstep_issue(crow, sidx_a, didx_a, bufs_a, gsem_a,
                                ssem_a, False)
            gb = halfstep_issue(crow + 4, sidx_b, didx_b, bufs_b, gsem_b,
                                ssem_b, False)
            halfstep_finish(ga, didx_a, bufs_a, ssem_a)
            halfstep_finish(gb, didx_b, bufs_b, ssem_b)

        @pl.when(b != 0)
        def _():
            ga = halfstep_issue(crow, sidx_a, didx_a, bufs_a, gsem_a,
                                ssem_a, True)
            gb = halfstep_issue(crow + 4, sidx_b, didx_b, bufs_b, gsem_b,
                                ssem_b, True)
            halfstep_finish(ga, didx_a, bufs_a, ssem_a)
            halfstep_finish(gb, didx_b, bufs_b, ssem_b)

        return carry

    lax.fori_loop(0, NCHUNK // NC // NS // 8, blk, 0)
    for j in range(4):
        pltpu.make_async_copy(bufs_a.at[j], acc_sh.at[didx_a.at[j]],
                              ssem_a.at[j]).wait()
        pltpu.make_async_copy(bufs_b.at[j], acc_sh.at[didx_b.at[j]],
                              ssem_b.at[j]).wait()
    plsc.subcore_barrier()

    @pl.when(c == 0)
    def _():
        pltpu.sync_copy(acc_sh.at[rows], out_hbm.at[rows, pl.ds(0, 8)])

    @pl.when(c != 0)
    def _():
        pltpu.sync_copy(acc_sh.at[rows], out_hbm.at[rows, pl.ds(8, 8)])


def _mm_body(x_ref, w_ref, o_ref):
    o_ref[...] = jnp.dot(x_ref[...], w_ref[...], preferred_element_type=jnp.float32)


def _matmul(x, w):
    m_blk = 2000
    return pl.pallas_call(
        _mm_body,
        grid=(N // m_blk,),
        in_specs=[
            pl.BlockSpec((m_blk, FP), lambda i: (i, 0)),
            pl.BlockSpec((FP, H), lambda i: (0, 0)),
        ],
        out_specs=pl.BlockSpec((m_blk, H), lambda i: (i, 0)),
        out_shape=jax.ShapeDtypeStruct((N, H), jnp.float32),
    )(x, w)


def _s1a_body(hist_ref, dinv_ref):
    deg = jnp.sum(hist_ref[...], axis=0) + 1.0
    dinv_ref[...] = lax.rsqrt(deg)


def _s1a_dinv(hist):
    return pl.pallas_call(
        _s1a_body,
        grid=(7,),
        in_specs=[pl.BlockSpec((NC * NS, NROW // 7, 128), lambda i: (0, i, 0))],
        out_specs=pl.BlockSpec((NROW // 7, 128), lambda i: (i, 0)),
        out_shape=jax.ShapeDtypeStruct((NROW, 128), jnp.float32),
    )(hist)


def _s1_body(xw_ref, dinv_ref, y_ref):
    y = xw_ref[...] * dinv_ref[...]
    y_ref[...] = jnp.pad(y, ((0, 0), (0, 128 - H)))


def _s1_scale(xw, dinv_col):
    return pl.pallas_call(
        _s1_body,
        grid=(NPAD // MBLK,),
        in_specs=[
            pl.BlockSpec((MBLK, H), lambda i: (i, 0)),
            pl.BlockSpec((MBLK, 1), lambda i: (i, 0)),
        ],
        out_specs=pl.BlockSpec((MBLK, 128), lambda i: (i, 0)),
        out_shape=jax.ShapeDtypeStruct((NPAD, 128), jnp.float32),
    )(xw, dinv_col)


def _s2_body(acc_ref, dinv_ref, w2_ref, b1_ref, y2_ref):
    acc = acc_ref[:, :H]
    dinv = dinv_ref[...]
    h = jax.nn.relu(acc * dinv + b1_ref[...])
    hw = jnp.dot(h, w2_ref[...], preferred_element_type=jnp.float32)
    y2_ref[...] = jnp.pad(hw * dinv, ((0, 0), (0, 128 - 8)))


def _s2_hidden(acc128, dinv_col, w2p, b1):
    return pl.pallas_call(
        _s2_body,
        grid=(NPAD // MBLK,),
        in_specs=[
            pl.BlockSpec((MBLK, 128), lambda i: (i, 0)),
            pl.BlockSpec((MBLK, 1), lambda i: (i, 0)),
            pl.BlockSpec((H, 8), lambda i: (0, 0)),
            pl.BlockSpec((1, H), lambda i: (0, 0)),
        ],
        out_specs=pl.BlockSpec((MBLK, 128), lambda i: (i, 0)),
        out_shape=jax.ShapeDtypeStruct((NPAD, 128), jnp.float32),
    )(acc128, dinv_col, w2p, b1)


def _s3_body(acc_ref, dinv_ref, b2_ref, o_ref):
    acc = acc_ref[:, 0:8] + acc_ref[:, 8:16]
    logits = acc * dinv_ref[...] + b2_ref[...]
    x7 = logits[:, :C]
    m = jnp.max(x7, axis=1, keepdims=True)
    lse = jnp.log(jnp.sum(jnp.exp(x7 - m), axis=1, keepdims=True)) + m
    o_ref[...] = jnp.pad(x7 - lse, ((0, 0), (0, 8 - C)))


def _s3_out(acc128, dinv_col, b2p):
    return pl.pallas_call(
        _s3_body,
        grid=(NPAD // MBLK,),
        in_specs=[
            pl.BlockSpec((MBLK, 128), lambda i: (i, 0)),
            pl.BlockSpec((MBLK, 1), lambda i: (i, 0)),
            pl.BlockSpec((1, 8), lambda i: (0, 0)),
        ],
        out_specs=pl.BlockSpec((MBLK, 8), lambda i: (i, 0)),
        out_shape=jax.ShapeDtypeStruct((NPAD, 8), jnp.float32),
    )(acc128, dinv_col, b2p)


def kernel(x, edge_index, W1, b1, W2, b2):
    src = edge_index[0].astype(jnp.int32)
    dst = edge_index[1].astype(jnp.int32)
    src2 = jnp.concatenate(
        [src, jnp.zeros((EPAD - E,), jnp.int32)]).reshape(NCHUNK, 128)
    dst2 = jnp.concatenate(
        [dst, jnp.full((EPAD - E,), NPAD - 1, jnp.int32)]).reshape(NCHUNK, 128)

    zeros8 = jnp.zeros((NPAD, 8), jnp.float32)

    hist = _k1_deg(dst2)
    dinv_col = _s1a_dinv(hist).reshape(NPAD, 1)

    xp = jnp.pad(x, ((0, 0), (0, FP - F)))
    w1p = jnp.pad(W1, ((0, FP - F), (0, 0)))
    xw = jnp.pad(_matmul(xp, w1p), ((0, NPAD - N), (0, 0)))

    y128 = _s1_scale(xw, dinv_col)
    acc128, _ = _k2_agg64(y128, src2, dst2)

    w2p = jnp.pad(W2, ((0, 0), (0, 8 - C)))
    y2_128 = _s2_hidden(acc128, dinv_col, w2p, b1.reshape(1, H))
    acc2_128, _ = _k3_agg16(y2_128, zeros8, src2, dst2)

    b2p = jnp.pad(b2, (0, 8 - C)).reshape(1, 8)
    outp = _s3_out(acc2_128, dinv_col, b2p)
    return outp[:N, :C]


# speedup vs baseline: 19.6208x; 1.8779x over previous
import functools

import jax
import jax.numpy as jnp
from jax import lax
from jax.experimental import pallas as pl
from jax.experimental.pallas import tpu as pltpu
from jax.experimental.pallas import tpu_sc as plsc

N = 50000
F = 1433
H = 64
C = 7
E = 800000

FP = 1536
NROW = 392
NPAD = NROW * 128
EPAD = 819200
NCHUNK = EPAD // 128
NC, NS = 2, 16
NSL = NPAD // NS
MBLK = NPAD // 8

_mesh = plsc.VectorSubcoreMesh(
    core_axis_name="c", subcore_axis_name="s", num_cores=NC, num_subcores=NS
)
_sc_params = pltpu.CompilerParams(use_tc_tiling_on_sc=False)
_sc_params_nolayout = pltpu.CompilerParams(
    use_tc_tiling_on_sc=False, needs_layout_passes=False)


@functools.partial(
    pl.kernel,
    out_type=jax.ShapeDtypeStruct((NC * NS, NROW, 128), jnp.float32),
    mesh=_mesh,
    compiler_params=_sc_params_nolayout,
    scratch_types=[
        pltpu.VMEM((8, 128), jnp.int32),
        pltpu.VMEM((NROW, 128), jnp.float32),
    ],
)
def _k1_deg(dst2_hbm, hist_hbm, idx_v, hist_v):
    c = lax.axis_index("c")
    s = lax.axis_index("s")
    wid = c * NS + s

    zeros = jnp.zeros((16,), jnp.float32)

    def zrow(r, carry):
        for k in range(8):
            hist_v[r, pl.ds(k * 16, 16)] = zeros
        return carry

    lax.fori_loop(0, NROW, zrow, 0)

    ones = jnp.ones((16,), jnp.float32)
    row0 = wid * (NCHUNK // (NC * NS))

    def blk(b, carry):
        crow = row0 + b * 8
        pltpu.sync_copy(dst2_hbm.at[pl.ds(crow, 8)], idx_v)
        for j in range(8):
            for k in range(8):
                dvec = idx_v[j, pl.ds(k * 16, 16)]
                r = lax.shift_right_logical(dvec, 7)
                cc = lax.bitwise_and(dvec, 127)
                plsc.addupdate_scatter(hist_v, [r, cc], ones)
        return carry

    lax.fori_loop(0, NCHUNK // (NC * NS) // 8, blk, 0)
    pltpu.sync_copy(hist_v, hist_hbm.at[wid])


@functools.partial(
    pl.kernel,
    out_type=[
        jax.ShapeDtypeStruct((NPAD, 128), jnp.float32),
        jax.ShapeDtypeStruct((NC, NPAD, 16), jnp.float32),
    ],
    mesh=_mesh,
    compiler_params=_sc_params,
    scratch_types=[
        pltpu.VMEM((4, 128), jnp.int32),
        pltpu.VMEM((4, 128), jnp.int32),
        pltpu.VMEM((4, 128), jnp.int32),
        pltpu.VMEM((4, 128), jnp.int32),
        pltpu.VMEM((4, 128, 16), jnp.float32),
        pltpu.VMEM((4, 128, 16), jnp.float32),
        pltpu.VMEM((NROW, 16), jnp.float32),
        pltpu.VMEM_SHARED((NPAD, 16), jnp.float32),
        pltpu.SemaphoreType.DMA((4,)),
        pltpu.SemaphoreType.DMA((4,)),
        pltpu.SemaphoreType.DMA((4,)),
        pltpu.SemaphoreType.DMA((4,)),
    ],
)
def _k2_agg64(y_hbm, src2_hbm, dst2_hbm, out_hbm, ycomp_hbm,
              sidx_a, didx_a, sidx_b, didx_b, bufs_a, bufs_b, stage_v, acc_sh,
              gsem_a, ssem_a, gsem_b, ssem_b):
    c = lax.axis_index("c")
    s = lax.axis_index("s")
    rbase = s * NSL
    rows = pl.ds(rbase, NSL)
    row0 = s * (NCHUNK // NS)
    tab = ycomp_hbm.at[c]

    for p in range(2):
        def compact(i, carry, p=p):
            rr = pl.ds(rbase + i * NROW, NROW)

            @pl.when(c == 0)
            def _():
                pltpu.sync_copy(y_hbm.at[rr, pl.ds(16 * p, 16)], stage_v)

            @pl.when(c != 0)
            def _():
                pltpu.sync_copy(y_hbm.at[rr, pl.ds(32 + 16 * p, 16)], stage_v)

            pltpu.sync_copy(stage_v, ycomp_hbm.at[c].at[rr])
            pltpu.sync_copy(stage_v, acc_sh.at[rr])
            return carry

        lax.fori_loop(0, NSL // NROW, compact, 0)
        plsc.subcore_barrier()

        def halfstep_issue(crow, sidx, didx, bufs, gsem, ssem, drain):
            if drain:
                for j in range(4):
                    pltpu.make_async_copy(
                        bufs.at[j], acc_sh.at[didx.at[j]], ssem.at[j]).wait()
            pltpu.sync_copy(src2_hbm.at[pl.ds(crow, 4)], sidx)
            pltpu.sync_copy(dst2_hbm.at[pl.ds(crow, 4)], didx)
            gds = []
            for j in range(4):
                d = pltpu.make_async_copy(tab.at[sidx.at[j]], bufs.at[j],
                                          gsem.at[j])
                d.start()
                gds.append(d)
            return gds

        def halfstep_finish(gds, didx, bufs, ssem):
            for j in range(4):
                gds[j].wait()
                d = pltpu.make_async_copy(bufs.at[j], acc_sh.at[didx.at[j]],
                                          ssem.at[j])
                d.start(add=True)

        def blk(b, carry):
            crow = row0 + b * 8

            @pl.when(b == 0)
            def _():
                ga = halfstep_issue(crow, sidx_a, didx_a, bufs_a, gsem_a,
                                    ssem_a, False)
                gb = halfstep_issue(crow + 4, sidx_b, didx_b, bufs_b, gsem_b,
                                    ssem_b, False)
                halfstep_finish(ga, didx_a, bufs_a, ssem_a)
                halfstep_finish(gb, didx_b, bufs_b, ssem_b)

            @pl.when(b != 0)
            def _():
                ga = halfstep_issue(crow, sidx_a, didx_a, bufs_a, gsem_a,
                                    ssem_a, True)
                gb = halfstep_issue(crow + 4, sidx_b, didx_b, bufs_b, gsem_b,
                                    ssem_b, True)
                halfstep_finish(ga, didx_a, bufs_a, ssem_a)
                halfstep_finish(gb, didx_b, bufs_b, ssem_b)

            return carry

        lax.fori_loop(0, NCHUNK // NS // 8, blk, 0)
        for j in range(4):
            pltpu.make_async_copy(bufs_a.at[j], acc_sh.at[didx_a.at[j]],
                                  ssem_a.at[j]).wait()
            pltpu.make_async_copy(bufs_b.at[j], acc_sh.at[didx_b.at[j]],
                                  ssem_b.at[j]).wait()
        plsc.subcore_barrier()

        @pl.when(c == 0)
        def _(p=p):
            pltpu.sync_copy(acc_sh.at[rows],
                            out_hbm.at[rows, pl.ds(16 * p, 16)])

        @pl.when(c != 0)
        def _(p=p):
            pltpu.sync_copy(acc_sh.at[rows],
                            out_hbm.at[rows, pl.ds(32 + 16 * p, 16)])

        plsc.subcore_barrier()


@functools.partial(
    pl.kernel,
    out_type=[
        jax.ShapeDtypeStruct((NPAD, 128), jnp.float32),
        jax.ShapeDtypeStruct((NC, NPAD, 8), jnp.float32),
    ],
    mesh=_mesh,
    compiler_params=_sc_params,
    scratch_types=[
        pltpu.VMEM((4, 128), jnp.int32),
        pltpu.VMEM((4, 128), jnp.int32),
        pltpu.VMEM((4, 128), jnp.int32),
        pltpu.VMEM((4, 128), jnp.int32),
        pltpu.VMEM((4, 128, 8), jnp.float32),
        pltpu.VMEM((4, 128, 8), jnp.float32),
        pltpu.VMEM((NROW, 8), jnp.float32),
        pltpu.VMEM_SHARED((NPAD, 8), jnp.float32),
        pltpu.SemaphoreType.DMA((4,)),
        pltpu.SemaphoreType.DMA((4,)),
        pltpu.SemaphoreType.DMA((4,)),
        pltpu.SemaphoreType.DMA((4,)),
    ],
)
def _k3_agg16(y2_hbm, zeros_hbm, src2_hbm, dst2_hbm, out_hbm, y2c_hbm,
              sidx_a, didx_a, sidx_b, didx_b, bufs_a, bufs_b, stage_v, acc_sh,
              gsem_a, ssem_a, gsem_b, ssem_b):
    c = lax.axis_index("c")
    s = lax.axis_index("s")
    rbase = s * NSL
    rows = pl.ds(rbase, NSL)

    def compact(i, carry):
        rr = pl.ds(rbase + i * NROW, NROW)
        pltpu.sync_copy(y2_hbm.at[rr, pl.ds(0, 8)], stage_v)
        pltpu.sync_copy(stage_v, y2c_hbm.at[c].at[rr])

        @pl.when(c == 0)
        def _():
            pltpu.sync_copy(stage_v, acc_sh.at[rr])

        return carry

    lax.fori_loop(0, NSL // NROW, compact, 0)

    @pl.when(c != 0)
    def _():
        pltpu.sync_copy(zeros_hbm.at[rows], acc_sh.at[rows])

    plsc.subcore_barrier()

    tab = y2c_hbm.at[c]
    row0 = c * (NCHUNK // NC) + s * (NCHUNK // NC // NS)

    def halfstep_issue(crow, sidx, didx, bufs, gsem, ssem, drain):
        if drain:
            for j in range(4):
                pltpu.make_async_copy(
                    bufs.at[j], acc_sh.at[didx.at[j]], ssem.at[j]).wait()
        pltpu.sync_copy(src2_hbm.at[pl.ds(crow, 4)], sidx)
        pltpu.sync_copy(dst2_hbm.at[pl.ds(crow, 4)], didx)
        gds = []
        for j in range(4):
            d = pltpu.make_async_copy(tab.at[sidx.at[j]], bufs.at[j], gsem.at[j])
            d.start()
            gds.append(d)
        return gds

    def halfstep_finish(gds, didx, bufs, ssem):
        for j in range(4):
            gds[j].wait()
            d = pltpu.make_async_copy(bufs.at[j], acc_sh.at[didx.at[j]],
                                      ssem.at[j])
            d.start(add=True)

    def blk(b, carry):
        crow = row0 + b * 8

        @pl.when(b == 0)
        def _():
            ga = halfstep_issue(crow, sidx_a, didx_a, bufs_a, gsem_a,
                                ssem_a, False)
            gb = halfstep_issue(crow + 4, sidx_b, didx_b, bufs_b, gsem_b,
                                ssem_b, False)
            halfstep_finish(ga, didx_a, bufs_a, ssem_a)
            halfstep_finish(gb, didx_b, bufs_b, ssem_b)

        @pl.when(b != 0)
        def _():
            ga = halfstep_issue(crow, sidx_a, didx_a, bufs_a, gsem_a,
                                ssem_a, True)
            gb = halfstep_issue(crow + 4, sidx_b, didx_b, bufs_b, gsem_b,
                                ssem_b, True)
            halfstep_finish(ga, didx_a, bufs_a, ssem_a)
            halfstep_finish(gb, didx_b, bufs_b, ssem_b)

        return carry

    lax.fori_loop(0, NCHUNK // NC // NS // 8, blk, 0)
    for j in range(4):
        pltpu.make_async_copy(bufs_a.at[j], acc_sh.at[didx_a.at[j]],
                              ssem_a.at[j]).wait()
        pltpu.make_async_copy(bufs_b.at[j], acc_sh.at[didx_b.at[j]],
                              ssem_b.at[j]).wait()
    plsc.subcore_barrier()

    @pl.when(c == 0)
    def _():
        pltpu.sync_copy(acc_sh.at[rows], out_hbm.at[rows, pl.ds(0, 8)])

    @pl.when(c != 0)
    def _():
        pltpu.sync_copy(acc_sh.at[rows], out_hbm.at[rows, pl.ds(8, 8)])


def _mm_body(x_ref, w_ref, o_ref):
    o_ref[...] = jnp.dot(x_ref[...], w_ref[...], preferred_element_type=jnp.float32)


def _matmul(x, w):
    m_blk = 2000
    return pl.pallas_call(
        _mm_body,
        grid=(N // m_blk,),
        in_specs=[
            pl.BlockSpec((m_blk, F), lambda i: (i, 0)),
            pl.BlockSpec((F, H), lambda i: (0, 0)),
        ],
        out_specs=pl.BlockSpec((m_blk, H), lambda i: (i, 0)),
        out_shape=jax.ShapeDtypeStruct((NPAD, H), jnp.float32),
    )(x, w)


def _s1a_body(hist_ref, dinv_ref):
    deg = jnp.sum(hist_ref[...], axis=0) + 1.0
    dinv_ref[...] = lax.rsqrt(deg)


def _s1a_dinv(hist):
    return pl.pallas_call(
        _s1a_body,
        grid=(7,),
        in_specs=[pl.BlockSpec((NC * NS, NROW // 7, 128), lambda i: (0, i, 0))],
        out_specs=pl.BlockSpec((NROW // 7, 128), lambda i: (i, 0)),
        out_shape=jax.ShapeDtypeStruct((NROW, 128), jnp.float32),
    )(hist)


def _s1_body(xw_ref, dinv_ref, y_ref):
    y = xw_ref[...] * dinv_ref[...]
    y_ref[...] = jnp.pad(y, ((0, 0), (0, 128 - H)))


def _s1_scale(xw, dinv_col):
    return pl.pallas_call(
        _s1_body,
        grid=(NPAD // MBLK,),
        in_specs=[
            pl.BlockSpec((MBLK, H), lambda i: (i, 0)),
            pl.BlockSpec((MBLK, 1), lambda i: (i, 0)),
        ],
        out_specs=pl.BlockSpec((MBLK, 128), lambda i: (i, 0)),
        out_shape=jax.ShapeDtypeStruct((NPAD, 128), jnp.float32),
    )(xw, dinv_col)


def _s2_body(acc_ref, dinv_ref, w2_ref, b1_ref, y2_ref):
    acc = acc_ref[:, :H]
    dinv = dinv_ref[...]
    h = jax.nn.relu(acc * dinv + b1_ref[...])
    hw = jnp.dot(h, w2_ref[...], preferred_element_type=jnp.float32)
    y2_ref[...] = jnp.pad(hw * dinv, ((0, 0), (0, 128 - 8)))


def _s2_hidden(acc128, dinv_col, w2p, b1):
    return pl.pallas_call(
        _s2_body,
        grid=(NPAD // MBLK,),
        in_specs=[
            pl.BlockSpec((MBLK, 128), lambda i: (i, 0)),
            pl.BlockSpec((MBLK, 1), lambda i: (i, 0)),
            pl.BlockSpec((H, 8), lambda i: (0, 0)),
            pl.BlockSpec((1, H), lambda i: (0, 0)),
        ],
        out_specs=pl.BlockSpec((MBLK, 128), lambda i: (i, 0)),
        out_shape=jax.ShapeDtypeStruct((NPAD, 128), jnp.float32),
    )(acc128, dinv_col, w2p, b1)


def _s3_body(acc_ref, dinv_ref, b2_ref, o_ref):
    acc = acc_ref[:, 0:8] + acc_ref[:, 8:16]
    logits = acc * dinv_ref[...] + b2_ref[...]
    x7 = logits[:, :C]
    m = jnp.max(x7, axis=1, keepdims=True)
    lse = jnp.log(jnp.sum(jnp.exp(x7 - m), axis=1, keepdims=True)) + m
    o_ref[...] = jnp.pad(x7 - lse, ((0, 0), (0, 8 - C)))


def _s3_out(acc128, dinv_col, b2p):
    return pl.pallas_call(
        _s3_body,
        grid=(NPAD // MBLK,),
        in_specs=[
            pl.BlockSpec((MBLK, 128), lambda i: (i, 0)),
            pl.BlockSpec((MBLK, 1), lambda i: (i, 0)),
            pl.BlockSpec((1, 8), lambda i: (0, 0)),
        ],
        out_specs=pl.BlockSpec((MBLK, 8), lambda i: (i, 0)),
        out_shape=jax.ShapeDtypeStruct((NPAD, 8), jnp.float32),
    )(acc128, dinv_col, b2p)


def kernel(x, edge_index, W1, b1, W2, b2):
    src = edge_index[0].astype(jnp.int32)
    dst = edge_index[1].astype(jnp.int32)
    src2 = jnp.concatenate(
        [src, jnp.zeros((EPAD - E,), jnp.int32)]).reshape(NCHUNK, 128)
    dst2 = jnp.concatenate(
        [dst, jnp.full((EPAD - E,), NPAD - 1, jnp.int32)]).reshape(NCHUNK, 128)

    zeros8 = jnp.zeros((NPAD, 8), jnp.float32)

    hist = _k1_deg(dst2)
    dinv_col = _s1a_dinv(hist).reshape(NPAD, 1)

    xw = _matmul(x, W1)

    y128 = _s1_scale(xw, dinv_col)
    acc128, _ = _k2_agg64(y128, src2, dst2)

    w2p = jnp.pad(W2, ((0, 0), (0, 8 - C)))
    y2_128 = _s2_hidden(acc128, dinv_col, w2p, b1.reshape(1, H))
    acc2_128, _ = _k3_agg16(y2_128, zeros8, src2, dst2)

    b2p = jnp.pad(b2, (0, 8 - C)).reshape(1, 8)
    outp = _s3_out(acc2_128, dinv_col, b2p)
    return outp[:N, :C]
